# baseline (speedup 1.0000x reference)
import functools

import jax
import jax.numpy as jnp
from jax import lax
from jax.experimental import pallas as pl
from jax.experimental.pallas import tpu as pltpu
from jax.experimental.pallas import tpu_sc as plsc


def _make_gather(B, T, D):
    mesh = plsc.ScalarSubcoreMesh(axis_name="c", num_cores=1)

    @functools.partial(
        pl.kernel,
        mesh=mesh,
        out_type=jax.ShapeDtypeStruct((B, D), jnp.float32),
        scratch_types=[
            pltpu.SMEM((B,), jnp.int32),
            pltpu.SemaphoreType.DMA,
        ],
    )
    def k(xf_hbm, len_hbm, out_hbm, len_s, sem):
        pltpu.sync_copy(len_hbm, len_s)
        cps = []
        for b in range(B):
            r = b * T + len_s[b] - 1
            cps.append(
                pltpu.async_copy(
                    xf_hbm.at[pl.ds(r, 1)], out_hbm.at[pl.ds(b, 1)], sem
                )
            )
        for cp in cps:
            cp.wait()

    return k


def kernel(x, lengths):
    B, T, D = x.shape
    xflat = x.reshape(B * T, D)
    return _make_gather(B, T, D)(xflat, lengths.astype(jnp.int32))


# speedup vs baseline: 1.0126x; 1.0126x over previous
import functools

import jax
import jax.numpy as jnp
from jax import lax
from jax.experimental import pallas as pl
from jax.experimental.pallas import tpu as pltpu
from jax.experimental.pallas import tpu_sc as plsc


def _make_gather(B, T, D):
    mesh = plsc.ScalarSubcoreMesh(axis_name="c", num_cores=1)

    @functools.partial(
        pl.kernel,
        mesh=mesh,
        out_type=jax.ShapeDtypeStruct((B, D), jnp.float32),
        scratch_types=[
            pltpu.SMEM((B,), jnp.int32),
            pltpu.SemaphoreType.DMA,
        ],
    )
    def k(xf_hbm, len_hbm, out_hbm, len_s, sem):
        pltpu.sync_copy(len_hbm, len_s)
        for b in range(B):
            r = b * T + len_s[b] - 1
            pltpu.async_copy(xf_hbm.at[pl.ds(r, 1)], out_hbm.at[pl.ds(b, 1)], sem)
        pltpu.make_async_copy(xf_hbm.at[pl.ds(0, B)], out_hbm, sem).wait()

    return k


def kernel(x, lengths):
    B, T, D = x.shape
    xflat = x.reshape(B * T, D)
    return _make_gather(B, T, D)(xflat, lengths.astype(jnp.int32))
